# baseline (speedup 1.0000x reference)
import functools

import jax
import jax.numpy as jnp
from jax import lax
from jax.experimental import pallas as pl
from jax.experimental.pallas import tpu as pltpu
from jax.experimental.pallas import tpu_sc as plsc

B = 4096
U, I, C = 8, 6, 3
D = 32
N = U * I * C
ROW = 3 * D
NC, NS = 2, 16
UH = U // NC
NH = UH * I * C
BW = B // NS
SPLIT = 24


def _cross_body(u_hbm, i_hbm, c_hbm, out_hbm, u_v, i_v, c_v,
                sem, sem_in, sem_d):
    cid = lax.axis_index("c")
    sid = lax.axis_index("s")
    b0 = sid * BW
    u0 = cid * UH
    n0 = u0 * I * C

    for m in range(SPLIT):
        n = n0 + m
        for p, src in enumerate((u_hbm.at[u0 + m // (I * C)],
                                 i_hbm.at[(m // C) % I],
                                 c_hbm.at[m % C])):
            t = (m * 3 + p) % NS

            @pl.when(sid == t)
            def _(n=n, p=p, src=src):
                pltpu.async_copy(src, out_hbm.at[n, pl.ds(p * D, D), :],
                                 sem_d)

    pltpu.async_copy(u_hbm.at[pl.ds(u0, UH), :, pl.ds(b0, BW)], u_v, sem_in)
    pltpu.async_copy(i_hbm.at[:, :, pl.ds(b0, BW)], i_v, sem_in)
    pltpu.async_copy(c_hbm.at[:, :, pl.ds(b0, BW)], c_v, sem_in)
    pltpu.make_async_copy(
        u_hbm.at[pl.ds(u0, UH), :, pl.ds(b0, BW)], u_v, sem_in).wait()
    pltpu.make_async_copy(i_hbm.at[:, :, pl.ds(b0, BW)], i_v, sem_in).wait()
    pltpu.make_async_copy(c_hbm.at[:, :, pl.ds(b0, BW)], c_v, sem_in).wait()

    for m in range(SPLIT, NH):
        n = n0 + m
        i = m // (I * C)
        j = (m // C) % I
        k = m % C
        pltpu.async_copy(
            u_v.at[i], out_hbm.at[n, pl.ds(0, D), pl.ds(b0, BW)], sem)
        pltpu.async_copy(
            i_v.at[j], out_hbm.at[n, pl.ds(D, D), pl.ds(b0, BW)], sem)
        pltpu.async_copy(
            c_v.at[k], out_hbm.at[n, pl.ds(2 * D, D), pl.ds(b0, BW)], sem)

    for m in range(SPLIT, NH):
        pltpu.make_async_copy(
            u_v.at[0], out_hbm.at[0, pl.ds(0, D), pl.ds(b0, BW)], sem).wait()
        pltpu.make_async_copy(
            i_v.at[0], out_hbm.at[0, pl.ds(D, D), pl.ds(b0, BW)], sem).wait()
        pltpu.make_async_copy(
            c_v.at[0], out_hbm.at[0, pl.ds(2 * D, D), pl.ds(b0, BW)],
            sem).wait()
    for m in range(SPLIT):
        for p in range(3):
            t = (m * 3 + p) % NS

            @pl.when(sid == t)
            def _():
                pltpu.make_async_copy(
                    u_hbm.at[0], out_hbm.at[0, pl.ds(0, D), :], sem_d).wait()


_cross_call = functools.partial(
    pl.kernel,
    out_type=jax.ShapeDtypeStruct((N, ROW, B), jnp.float32),
    mesh=plsc.VectorSubcoreMesh(
        core_axis_name="c", subcore_axis_name="s",
        num_cores=NC, num_subcores=NS),
    compiler_params=pltpu.CompilerParams(use_tc_tiling_on_sc=True),
    scratch_types=[
        pltpu.VMEM((UH, D, BW), jnp.float32),
        pltpu.VMEM((I, D, BW), jnp.float32),
        pltpu.VMEM((C, D, BW), jnp.float32),
        pltpu.SemaphoreType.DMA,
        pltpu.SemaphoreType.DMA,
        pltpu.SemaphoreType.DMA,
    ],
)(_cross_body)


@jax.jit
def kernel(x_user, x_item, x_context):
    out3 = _cross_call(
        jnp.transpose(x_user, (1, 2, 0)),
        jnp.transpose(x_item, (1, 2, 0)),
        jnp.transpose(x_context, (1, 2, 0)),
    )
    return jnp.transpose(out3, (2, 0, 1))


# speedup vs baseline: 23.8961x; 23.8961x over previous
import functools

import jax
import jax.numpy as jnp
from jax import lax
from jax.experimental import pallas as pl
from jax.experimental.pallas import tpu as pltpu
from jax.experimental.pallas import tpu_sc as plsc

B = 4096
U, I, C = 8, 6, 3
D = 32
N = U * I * C
ROW = 3 * D
NC, NS = 2, 16
UH = U // NC
BW = B // NS


def _cross_body(u_hbm, i_hbm, c_hbm, out_hbm, u_v, i_v, c_v, sem, sem_in):
    cid = lax.axis_index("c")
    sid = lax.axis_index("s")
    b0 = sid * BW
    u0 = cid * UH
    n0 = u0 * I * C

    pltpu.async_copy(u_hbm.at[pl.ds(u0, UH), :, pl.ds(b0, BW)], u_v, sem_in)
    pltpu.async_copy(i_hbm.at[:, :, pl.ds(b0, BW)], i_v, sem_in)
    pltpu.async_copy(c_hbm.at[:, :, pl.ds(b0, BW)], c_v, sem_in)

    pltpu.make_async_copy(
        u_hbm.at[pl.ds(u0, UH), :, pl.ds(b0, BW)], u_v, sem_in).wait()
    for i in range(UH):
        for m in range(I * C):
            n = n0 + i * I * C + m
            pltpu.async_copy(
                u_v.at[i], out_hbm.at[n, pl.ds(0, D), pl.ds(b0, BW)], sem)
    pltpu.make_async_copy(i_hbm.at[:, :, pl.ds(b0, BW)], i_v, sem_in).wait()
    for j in range(I):
        for i in range(UH):
            for k in range(C):
                n = n0 + (i * I + j) * C + k
                pltpu.async_copy(
                    i_v.at[j], out_hbm.at[n, pl.ds(D, D), pl.ds(b0, BW)], sem)
    pltpu.make_async_copy(c_hbm.at[:, :, pl.ds(b0, BW)], c_v, sem_in).wait()
    for k in range(C):
        for m in range(UH * I):
            n = n0 + m * C + k
            pltpu.async_copy(
                c_v.at[k], out_hbm.at[n, pl.ds(2 * D, D), pl.ds(b0, BW)], sem)
    for _ in range(UH * I * C):
        pltpu.make_async_copy(
            u_v.at[0], out_hbm.at[0, pl.ds(0, D), pl.ds(b0, BW)], sem).wait()
        pltpu.make_async_copy(
            i_v.at[0], out_hbm.at[0, pl.ds(D, D), pl.ds(b0, BW)], sem).wait()
        pltpu.make_async_copy(
            c_v.at[0], out_hbm.at[0, pl.ds(2 * D, D), pl.ds(b0, BW)],
            sem).wait()


_cross_call = functools.partial(
    pl.kernel,
    out_type=jax.ShapeDtypeStruct((N, ROW, B), jnp.float32),
    mesh=plsc.VectorSubcoreMesh(
        core_axis_name="c", subcore_axis_name="s",
        num_cores=NC, num_subcores=NS),
    compiler_params=pltpu.CompilerParams(use_tc_tiling_on_sc=True),
    scratch_types=[
        pltpu.VMEM((UH, D, BW), jnp.float32),
        pltpu.VMEM((I, D, BW), jnp.float32),
        pltpu.VMEM((C, D, BW), jnp.float32),
        pltpu.SemaphoreType.DMA,
        pltpu.SemaphoreType.DMA,
    ],
)(_cross_body)


@jax.jit
def kernel(x_user, x_item, x_context):
    out3 = _cross_call(
        jnp.transpose(x_user, (1, 2, 0)),
        jnp.transpose(x_item, (1, 2, 0)),
        jnp.transpose(x_context, (1, 2, 0)),
    )
    return jnp.transpose(out3, (2, 0, 1))
